# baseline (speedup 1.0000x reference)
import functools

import jax
import jax.numpy as jnp
from jax import lax
from jax.experimental import pallas as pl
from jax.experimental.pallas import tpu as pltpu
from jax.experimental.pallas import tpu_sc as plsc

_LOG2 = 0.6931471805599453

_NUM_CORES = 2
_NUM_SUBCORES = 16

_IDXW = 128
_IROWS = 12
_BE = 256


def _ssp(x):
    return jnp.maximum(x, 0.0) + jnp.log1p(jnp.exp(-jnp.abs(x))) - _LOG2


def _edge_kernel(xl_ref, xh_ref, wa_ref, ba_ref, wb_ref, bb_ref, o_ref):
    def mlp(x):
        z = _ssp(jnp.dot(x, wa_ref[...], preferred_element_type=jnp.float32)
                 + ba_ref[...])
        return jnp.dot(z, wb_ref[...], preferred_element_type=jnp.float32) + bb_ref[...]

    o_ref[...] = jnp.concatenate([mlp(xl_ref[...]), mlp(xh_ref[...])], axis=1)


def _edge_mlp(ef, wa_t, ba, wb_t, bb, eh):
    e, dim = ef.shape
    grid = eh // _BE
    last_real_hi = e // _BE - 1

    return pl.pallas_call(
        _edge_kernel,
        grid=(grid,),
        in_specs=[
            pl.BlockSpec((_BE, dim), lambda b: (b, 0)),
            pl.BlockSpec(
                (_BE, dim),
                lambda b: (jnp.minimum(eh // _BE + b, last_real_hi), 0),
            ),
            pl.BlockSpec(wa_t.shape, lambda b: (0, 0)),
            pl.BlockSpec((1, dim), lambda b: (0, 0)),
            pl.BlockSpec(wb_t.shape, lambda b: (0, 0)),
            pl.BlockSpec((1, dim), lambda b: (0, 0)),
        ],
        out_specs=pl.BlockSpec((_BE, 2 * dim), lambda b: (b, 0)),
        out_shape=jax.ShapeDtypeStruct((eh, 2 * dim), jnp.float32),
        compiler_params=pltpu.CompilerParams(dimension_semantics=("parallel",)),
    )(ef, ef, wa_t, ba.reshape(1, dim), wb_t, bb.reshape(1, dim))


def _node_kernel(x_ref, wa_ref, ba_ref, wb_ref, bb_ref, o_ref):
    z = _ssp(jnp.dot(x_ref[...], wa_ref[...], preferred_element_type=jnp.float32)
             + ba_ref[...])
    o_ref[...] = (
        jnp.dot(z, wb_ref[...], preferred_element_type=jnp.float32) + bb_ref[...]
    )


def _node_mlp(x, wa_t, ba, wb_t, bb, block_rows):
    rows, dim = x.shape
    return pl.pallas_call(
        _node_kernel,
        grid=(rows // block_rows,),
        in_specs=[
            pl.BlockSpec((block_rows, dim), lambda i: (i, 0)),
            pl.BlockSpec(wa_t.shape, lambda i: (0, 0)),
            pl.BlockSpec((1, dim), lambda i: (0, 0)),
            pl.BlockSpec(wb_t.shape, lambda i: (0, 0)),
            pl.BlockSpec((1, dim), lambda i: (0, 0)),
        ],
        out_specs=pl.BlockSpec((block_rows, dim), lambda i: (i, 0)),
        out_shape=jax.ShapeDtypeStruct((rows, dim), jnp.float32),
        compiler_params=pltpu.CompilerParams(dimension_semantics=("parallel",)),
    )(x, wa_t, ba.reshape(1, dim), wb_t, bb.reshape(1, dim))


def _seg_sum(he2, dstp, n, dim):
    half = n // _NUM_CORES
    half_pad = half + 88
    n_chunks = dstp.shape[0]
    rows_per_sub = n_chunks // _NUM_SUBCORES
    wb_chunk = 200
    n_wb = half // wb_chunk
    mesh = plsc.VectorSubcoreMesh(core_axis_name="c", subcore_axis_name="s")

    @functools.partial(
        pl.kernel,
        mesh=mesh,
        compiler_params=pltpu.CompilerParams(use_tc_tiling_on_sc=False),
        out_type=jax.ShapeDtypeStruct((n, dim), jnp.float32),
        scratch_types=[
            pltpu.VMEM((_IROWS, _IDXW), jnp.int32),
            pltpu.VMEM((_IROWS, _IDXW), jnp.int32),
            pltpu.VMEM((_IDXW, dim), jnp.float32),
            pltpu.VMEM((_IDXW, dim), jnp.float32),
            pltpu.VMEM_SHARED((half_pad, dim), jnp.float32),
            pltpu.SemaphoreType.DMA,
            pltpu.SemaphoreType.DMA,
        ],
    )
    def seg_kernel(he_hbm, dst_hbm, h_hbm, idx_raw, idx_loc, buf0, buf1, acc,
                   sem0, sem1):
        c = lax.axis_index("c")
        s = lax.axis_index("s")
        base_node = c * half
        bufs = (buf0, buf1)
        sems = (sem0, sem1)
        chunk0 = s * rows_per_sub

        def start_load(j, b):
            pltpu.async_copy(
                he_hbm.at[pl.ds(j * 64, 64), pl.ds(0, dim)],
                bufs[b].at[pl.ds(0, 64)],
                sems[b],
            )
            pltpu.async_copy(
                he_hbm.at[pl.ds(j * 64, 64), pl.ds(dim, dim)],
                bufs[b].at[pl.ds(64, 64)],
                sems[b],
            )

        def wait_load(j, b):
            pltpu.make_async_copy(
                he_hbm.at[pl.ds(j * 64, 64), pl.ds(0, dim)],
                bufs[b].at[pl.ds(0, 64)],
                sems[b],
            ).wait()
            pltpu.make_async_copy(
                he_hbm.at[pl.ds(j * 64, 64), pl.ds(dim, dim)],
                bufs[b].at[pl.ds(64, 64)],
                sems[b],
            ).wait()

        @pl.loop(0, _IDXW)
        def _(r):
            for g in range(dim // 16):
                buf0[r, pl.ds(g * 16, 16)] = jnp.zeros((16,), jnp.float32)

        init_rows = half_pad // _NUM_SUBCORES
        @pl.loop(0, 14)
        def _(k):
            pltpu.sync_copy(
                buf0.at[pl.ds(0, 112)],
                acc.at[pl.ds(s * init_rows + k * 112, 112)],
            )
        plsc.subcore_barrier()

        start_load(chunk0, 0)
        start_load(chunk0 + 1, 1)

        @pl.loop(0, rows_per_sub, step=_IROWS)
        def _(i):
            pltpu.sync_copy(dst_hbm.at[pl.ds(chunk0 + i, _IROWS)], idx_raw)
            for r in range(_IROWS):
                for g in range(_IDXW // 16):
                    v = idx_raw[r, pl.ds(g * 16, 16)]
                    lo = v - base_node
                    ok = (lo >= 0) & (lo < half)
                    idx_loc[r, pl.ds(g * 16, 16)] = jnp.where(ok, lo, half)
            for k in range(_IROWS):
                b = k % 2
                wait_load(chunk0 + i + k, b)
                pltpu.sync_copy(bufs[b], acc.at[idx_loc.at[k]], add=True)

                @pl.when(i + k + 2 < rows_per_sub)
                def _():
                    start_load(chunk0 + i + k + 2, b)

        plsc.subcore_barrier()

        @pl.loop(s, n_wb, step=_NUM_SUBCORES)
        def _(k):
            pltpu.sync_copy(
                acc.at[pl.ds(k * wb_chunk, wb_chunk)],
                h_hbm.at[pl.ds(base_node + k * wb_chunk, wb_chunk)],
            )

    return seg_kernel(he2, dstp)


def kernel(node_feats, edge_feats, edge_index, W1, b1, We1, be1, We2, be2,
           Wn1, bn1, Wn2, bn2):
    n, dim = node_feats.shape
    e = edge_feats.shape[0]

    e_pad = ((e + 24575) // 24576) * 24576
    eh = e_pad // 2

    dst = edge_index[1]
    dst_pad = jnp.concatenate([dst, jnp.full((e_pad - e,), n, jnp.int32)])
    dstp = jnp.concatenate(
        [dst_pad[:eh].reshape(-1, 64), dst_pad[eh:].reshape(-1, 64)], axis=1
    )

    he2 = _edge_mlp(edge_feats, We1.T, be1, We2.T, be2, eh)
    h = _seg_sum(he2, dstp, n, dim)
    return _node_mlp(h, Wn1.T, bn1, Wn2.T, bn2, block_rows=2000)


# speedup vs baseline: 1.7233x; 1.7233x over previous
import functools

import jax
import jax.numpy as jnp
from jax import lax
from jax.experimental import pallas as pl
from jax.experimental.pallas import tpu as pltpu
from jax.experimental.pallas import tpu_sc as plsc

_LOG2 = 0.6931471805599453

_NUM_CORES = 2
_NUM_SUBCORES = 16

_IDXW = 128
_IROWS = 10
_BE = 6400


def _ssp(x):
    return jnp.maximum(x, 0.0) + jnp.log1p(jnp.exp(-jnp.abs(x))) - _LOG2


def _edge_kernel(xl_ref, xh_ref, wa_ref, ba_ref, wb_ref, bb_ref, o_ref):
    def mlp(x):
        z = _ssp(jnp.dot(x, wa_ref[...], preferred_element_type=jnp.float32)
                 + ba_ref[...])
        return jnp.dot(z, wb_ref[...], preferred_element_type=jnp.float32) + bb_ref[...]

    o_ref[...] = jnp.concatenate([mlp(xl_ref[...]), mlp(xh_ref[...])], axis=1)


def _edge_mlp(ef, wa_t, ba, wb_t, bb, eh):
    e, dim = ef.shape
    grid = eh // _BE
    last_real = e // _BE - 1

    return pl.pallas_call(
        _edge_kernel,
        grid=(grid,),
        in_specs=[
            pl.BlockSpec((_BE, dim), lambda b: (b, 0)),
            pl.BlockSpec(
                (_BE, dim),
                lambda b: (jnp.minimum(eh // _BE + b, last_real), 0),
            ),
            pl.BlockSpec(wa_t.shape, lambda b: (0, 0)),
            pl.BlockSpec((1, dim), lambda b: (0, 0)),
            pl.BlockSpec(wb_t.shape, lambda b: (0, 0)),
            pl.BlockSpec((1, dim), lambda b: (0, 0)),
        ],
        out_specs=pl.BlockSpec((_BE, 2 * dim), lambda b: (b, 0)),
        out_shape=jax.ShapeDtypeStruct((eh, 2 * dim), jnp.float32),
        compiler_params=pltpu.CompilerParams(dimension_semantics=("parallel",)),
    )(ef, ef, wa_t, ba.reshape(1, dim), wb_t, bb.reshape(1, dim))


def _node_kernel(x_ref, wa_ref, ba_ref, wb_ref, bb_ref, o_ref):
    z = _ssp(jnp.dot(x_ref[...], wa_ref[...], preferred_element_type=jnp.float32)
             + ba_ref[...])
    o_ref[...] = (
        jnp.dot(z, wb_ref[...], preferred_element_type=jnp.float32) + bb_ref[...]
    )


def _node_mlp(x, wa_t, ba, wb_t, bb, block_rows):
    rows, dim = x.shape
    return pl.pallas_call(
        _node_kernel,
        grid=(rows // block_rows,),
        in_specs=[
            pl.BlockSpec((block_rows, dim), lambda i: (i, 0)),
            pl.BlockSpec(wa_t.shape, lambda i: (0, 0)),
            pl.BlockSpec((1, dim), lambda i: (0, 0)),
            pl.BlockSpec(wb_t.shape, lambda i: (0, 0)),
            pl.BlockSpec((1, dim), lambda i: (0, 0)),
        ],
        out_specs=pl.BlockSpec((block_rows, dim), lambda i: (i, 0)),
        out_shape=jax.ShapeDtypeStruct((rows, dim), jnp.float32),
        compiler_params=pltpu.CompilerParams(dimension_semantics=("parallel",)),
    )(x, wa_t, ba.reshape(1, dim), wb_t, bb.reshape(1, dim))


def _seg_sum(he2, dstp, n, dim):
    half = n // _NUM_CORES
    half_pad = half + 88
    n_chunks = dstp.shape[0]
    rows_per_sub = n_chunks // _NUM_SUBCORES
    wb_chunk = 200
    n_wb = half // wb_chunk
    mesh = plsc.VectorSubcoreMesh(core_axis_name="c", subcore_axis_name="s")

    @functools.partial(
        pl.kernel,
        mesh=mesh,
        compiler_params=pltpu.CompilerParams(use_tc_tiling_on_sc=False),
        out_type=jax.ShapeDtypeStruct((n, dim), jnp.float32),
        scratch_types=[
            pltpu.VMEM((_IROWS, _IDXW), jnp.int32),
            pltpu.VMEM((_IROWS, _IDXW), jnp.int32),
            pltpu.VMEM((_IDXW, dim), jnp.float32),
            pltpu.VMEM((_IDXW, dim), jnp.float32),
            pltpu.VMEM_SHARED((half_pad, dim), jnp.float32),
            pltpu.SemaphoreType.DMA,
            pltpu.SemaphoreType.DMA,
        ],
    )
    def seg_kernel(he_hbm, dst_hbm, h_hbm, idx_raw, idx_loc, buf0, buf1, acc,
                   sem0, sem1):
        c = lax.axis_index("c")
        s = lax.axis_index("s")
        base_node = c * half
        bufs = (buf0, buf1)
        sems = (sem0, sem1)
        chunk0 = s * rows_per_sub

        def start_load(j, b):
            pltpu.async_copy(
                he_hbm.at[pl.ds(j * 64, 64), pl.ds(0, dim)],
                bufs[b].at[pl.ds(0, 64)],
                sems[b],
            )
            pltpu.async_copy(
                he_hbm.at[pl.ds(j * 64, 64), pl.ds(dim, dim)],
                bufs[b].at[pl.ds(64, 64)],
                sems[b],
            )

        def wait_load(j, b):
            pltpu.make_async_copy(
                he_hbm.at[pl.ds(j * 64, 64), pl.ds(0, dim)],
                bufs[b].at[pl.ds(0, 64)],
                sems[b],
            ).wait()
            pltpu.make_async_copy(
                he_hbm.at[pl.ds(j * 64, 64), pl.ds(dim, dim)],
                bufs[b].at[pl.ds(64, 64)],
                sems[b],
            ).wait()

        @pl.loop(0, _IDXW)
        def _(r):
            for g in range(dim // 16):
                buf0[r, pl.ds(g * 16, 16)] = jnp.zeros((16,), jnp.float32)

        init_rows = half_pad // _NUM_SUBCORES
        @pl.loop(0, 14)
        def _(k):
            pltpu.sync_copy(
                buf0.at[pl.ds(0, 112)],
                acc.at[pl.ds(s * init_rows + k * 112, 112)],
            )
        plsc.subcore_barrier()

        start_load(chunk0, 0)
        start_load(chunk0 + 1, 1)

        @pl.loop(0, rows_per_sub, step=_IROWS)
        def _(i):
            pltpu.sync_copy(dst_hbm.at[pl.ds(chunk0 + i, _IROWS)], idx_raw)
            for r in range(_IROWS):
                for g in range(_IDXW // 16):
                    v = idx_raw[r, pl.ds(g * 16, 16)]
                    lo = v - base_node
                    ok = (lo >= 0) & (lo < half)
                    idx_loc[r, pl.ds(g * 16, 16)] = jnp.where(ok, lo, half)
            for k in range(_IROWS):
                b = k % 2
                wait_load(chunk0 + i + k, b)
                pltpu.sync_copy(bufs[b], acc.at[idx_loc.at[k]], add=True)

                @pl.when(i + k + 2 < rows_per_sub)
                def _():
                    start_load(chunk0 + i + k + 2, b)

        plsc.subcore_barrier()

        @pl.loop(s, n_wb, step=_NUM_SUBCORES)
        def _(k):
            pltpu.sync_copy(
                acc.at[pl.ds(k * wb_chunk, wb_chunk)],
                h_hbm.at[pl.ds(base_node + k * wb_chunk, wb_chunk)],
            )

    return seg_kernel(he2, dstp)


def kernel(node_feats, edge_feats, edge_index, W1, b1, We1, be1, We2, be2,
           Wn1, bn1, Wn2, bn2):
    n, dim = node_feats.shape
    e = edge_feats.shape[0]

    e_pad = ((e + 102399) // 102400) * 102400
    eh = e_pad // 2

    dst = edge_index[1]
    dst_pad = jnp.concatenate([dst, jnp.full((e_pad - e,), n, jnp.int32)])
    dstp = jnp.concatenate(
        [dst_pad[:eh].reshape(-1, 64), dst_pad[eh:].reshape(-1, 64)], axis=1
    )

    he2 = _edge_mlp(edge_feats, We1.T, be1, We2.T, be2, eh)
    h = _seg_sum(he2, dstp, n, dim)
    return _node_mlp(h, Wn1.T, bn1, Wn2.T, bn2, block_rows=2000)
